# baseline (speedup 1.0000x reference)
import jax
import jax.numpy as jnp
from jax import lax
from jax.experimental import pallas as pl
from jax.experimental.pallas import tpu as pltpu

_V = 100000
_D = 300
_G = 100
_B = 100
_BV = 5000
_NB = _V // _BV

_SHIFT = 1 << 20
_MASK = _SHIFT - 1


def _dist_body(wgt_ref, wbt_ref, x_ref, counts_out):
    x = x_ref[...]
    wgt = wgt_ref[...]
    wbt = wbt_ref[...]
    pg = jnp.dot(x, wgt, preferred_element_type=jnp.float32)
    pb = jnp.dot(x, wbt, preferred_element_type=jnp.float32)
    g2 = jnp.sum(wgt * wgt, axis=0, keepdims=True)
    b2 = jnp.sum(wbt * wbt, axis=0, keepdims=True)
    tg = g2 - 2.0 * pg
    tb = b2 - 2.0 * pb
    m = jnp.min(tb, axis=1, keepdims=True)
    counts = jnp.sum((tg < m).astype(jnp.int32), axis=1, keepdims=True)
    counts_out[...] = counts[None]


_AR = 100
_AC = _V // _AR


def _argmax_body(counts_ref, idx_out, clue_out):
    counts = counts_ref[...]
    rows = (lax.broadcasted_iota(jnp.int32, (_AR, _AC), 0) * _AC
            + lax.broadcasted_iota(jnp.int32, (_AR, _AC), 1))
    combined = counts * _SHIFT + (_MASK - rows)
    best = jnp.max(combined)
    clue_out[0, 0] = best // _SHIFT
    idx_out[0, 0] = _MASK - (best & _MASK)


def _distance_pass(wgt, wbt, embeddings, interpret=False):
    return pl.pallas_call(
        _dist_body,
        grid=(_NB,),
        in_specs=[
            pl.BlockSpec((_D, _G), lambda i: (0, 0)),
            pl.BlockSpec((_D, _B), lambda i: (0, 0)),
            pl.BlockSpec((_BV, _D), lambda i: (i, 0)),
        ],
        out_specs=pl.BlockSpec((1, _BV, 1), lambda i: (i, 0, 0)),
        out_shape=jax.ShapeDtypeStruct((_NB, _BV, 1), jnp.int32),
        compiler_params=pltpu.CompilerParams(
            dimension_semantics=("parallel",),
        ),
        interpret=interpret,
    )(wgt, wbt, embeddings)


def _argmax_pass(counts, interpret=False):
    return pl.pallas_call(
        _argmax_body,
        out_specs=[
            pl.BlockSpec(memory_space=pltpu.SMEM),
            pl.BlockSpec(memory_space=pltpu.SMEM),
        ],
        out_shape=[
            jax.ShapeDtypeStruct((1, 1), jnp.int32),
            jax.ShapeDtypeStruct((1, 1), jnp.int32),
        ],
        interpret=interpret,
    )(counts)


def kernel(embeddings, good_idx, bad_idx):
    wgt = jnp.take(embeddings, good_idx.astype(jnp.int32), axis=0).T
    wbt = jnp.take(embeddings, bad_idx.astype(jnp.int32), axis=0).T
    counts = _distance_pass(wgt, wbt, embeddings).reshape(_V)
    idx, clue = _argmax_pass(counts.reshape(_AR, _AC))
    return (idx[0, 0], clue[0, 0], counts)
